# baseline (speedup 1.0000x reference)
import functools

import jax
import jax.numpy as jnp
from jax import lax
from jax.experimental import pallas as pl
from jax.experimental.pallas import tpu as pltpu
from jax.experimental.pallas import tpu_sc as plsc

DIM = 64
PITCH = 128
NUM_CORES = 2
NUM_SUBCORES = 16
NUM_WORKERS = NUM_CORES * NUM_SUBCORES
NBUF = 4
CHUNK = 400


def kernel(x, table):
    batch, seq = x.shape
    n_flat = batch * seq
    n_rows = n_flat // CHUNK
    rows_per_w = n_rows // NUM_WORKERS
    n_groups = rows_per_w // NBUF
    assert n_groups * NBUF == rows_per_w
    mesh = plsc.VectorSubcoreMesh(core_axis_name="c", subcore_axis_name="s")

    @functools.partial(
        pl.kernel,
        mesh=mesh,
        out_type=jax.ShapeDtypeStruct((n_flat, PITCH), jnp.float32),
        scratch_types=[
            pltpu.VMEM((rows_per_w, CHUNK), jnp.int32),
            pltpu.VMEM((NBUF, CHUNK, DIM), jnp.float32),
            pltpu.SemaphoreType.DMA((NBUF,)),
            pltpu.SemaphoreType.DMA((NBUF,)),
        ],
        compiler_params=pltpu.CompilerParams(use_tc_tiling_on_sc=False),
    )
    def k(x_hbm, table_hbm, out_hbm, idx_v, rows_v, gsem, ssem):
        wid = lax.axis_index("s") * NUM_CORES + lax.axis_index("c")
        row0 = wid * rows_per_w
        flat0 = row0 * CHUNK
        pltpu.sync_copy(x_hbm.at[pl.ds(row0, rows_per_w)], idx_v)

        def gather_copy(i, b):
            r = i * NBUF + b
            return pltpu.make_async_copy(
                table_hbm.at[idx_v.at[r]],
                rows_v.at[b],
                gsem.at[b],
            )

        def store_copy(i, b):
            r = i * NBUF + b
            return pltpu.make_async_copy(
                rows_v.at[b],
                out_hbm.at[pl.ds(flat0 + r * CHUNK, CHUNK), pl.ds(0, DIM)],
                ssem.at[b],
            )

        for b in range(NBUF):
            gather_copy(0, b).start()

        def body(i, carry):
            for b in range(NBUF):
                gather_copy(i, b).wait()
                store_copy(i, b).start()
            for b in range(NBUF):
                store_copy(i, b).wait()
                gather_copy(i + 1, b).start()
            return carry

        lax.fori_loop(0, n_groups - 1, body, 0)

        last = n_groups - 1
        for b in range(NBUF):
            gather_copy(last, b).wait()
            store_copy(last, b).start()
        for b in range(NBUF):
            store_copy(last, b).wait()

    padded = k(x.reshape(n_rows, CHUNK), table)
    return padded[:, :DIM].reshape(batch, seq, DIM)


# speedup vs baseline: 1.0025x; 1.0025x over previous
import functools

import jax
import jax.numpy as jnp
from jax import lax
from jax.experimental import pallas as pl
from jax.experimental.pallas import tpu as pltpu
from jax.experimental.pallas import tpu_sc as plsc

DIM = 64
PITCH = 128
NUM_CORES = 2
NUM_SUBCORES = 16
NUM_WORKERS = NUM_CORES * NUM_SUBCORES
NBUF = 4
PAIR = 1


def kernel(x, table):
    batch, seq = x.shape
    rows_per_w = batch // NUM_WORKERS
    n_slots = rows_per_w // PAIR
    n_groups = n_slots // NBUF
    assert n_groups * NBUF * PAIR == rows_per_w
    n_flat = batch * seq
    mesh = plsc.VectorSubcoreMesh(core_axis_name="c", subcore_axis_name="s")

    @functools.partial(
        pl.kernel,
        mesh=mesh,
        out_type=jax.ShapeDtypeStruct((n_flat, PITCH), jnp.float32),
        scratch_types=[
            pltpu.VMEM((rows_per_w, seq), jnp.int32),
            pltpu.VMEM((NBUF, PAIR * seq, DIM), jnp.float32),
            pltpu.SemaphoreType.DMA((NBUF,)),
            pltpu.SemaphoreType.DMA((NBUF,)),
        ],
        compiler_params=pltpu.CompilerParams(use_tc_tiling_on_sc=False),
    )
    def k(x_hbm, table_hbm, out_hbm, idx_v, rows_v, gsem, ssem):
        wid = lax.axis_index("s") * NUM_CORES + lax.axis_index("c")
        row0 = wid * rows_per_w
        flat0 = row0 * seq
        pltpu.sync_copy(x_hbm.at[pl.ds(row0, rows_per_w)], idx_v)

        def gather_copies(i, b):
            p = i * NBUF + b
            return [
                pltpu.make_async_copy(
                    table_hbm.at[idx_v.at[PAIR * p + j]],
                    rows_v.at[b, pl.ds(j * seq, seq)],
                    gsem.at[b],
                )
                for j in range(PAIR)
            ]

        def store_copy(i, b):
            p = i * NBUF + b
            return pltpu.make_async_copy(
                rows_v.at[b],
                out_hbm.at[
                    pl.ds(flat0 + p * PAIR * seq, PAIR * seq), pl.ds(0, DIM)
                ],
                ssem.at[b],
            )

        def start_gathers(i, b):
            for c in gather_copies(i, b):
                c.start()

        def wait_gathers(i, b):
            for c in gather_copies(i, b):
                c.wait()

        for b in range(NBUF):
            start_gathers(0, b)

        def body(i, carry):
            for b in range(NBUF):
                wait_gathers(i, b)
                store_copy(i, b).start()
            for b in range(NBUF):
                store_copy(i, b).wait()
                start_gathers(i + 1, b)
            return carry

        lax.fori_loop(0, n_groups - 1, body, 0)

        last = n_groups - 1
        for b in range(NBUF):
            wait_gathers(last, b)
            store_copy(last, b).start()
        for b in range(NBUF):
            store_copy(last, b).wait()

    padded = k(x, table)
    return padded[:, :DIM].reshape(batch, seq, DIM)
